# baseline (speedup 1.0000x reference)
import functools

import jax
import jax.numpy as jnp
from jax import lax
from jax.experimental import pallas as pl
from jax.experimental.pallas import tpu as pltpu
from jax.experimental.pallas import tpu_sc as plsc

N_NODES = 10000
D = 128
K = 3

NP = 10240
EP = 327680
EROWS = EP // 128
ROWS_T32 = EROWS // 32
NROW_T = NP // 16
IB = 64
EB = 8
NBLK = ROWS_T32 // EB


def _deg_body(dst_hbm, w_hbm, out_hbm, dst_v, w_v, deg_v):
    c = lax.axis_index("c")
    s = lax.axis_index("s")
    wid = c * 16 + s
    base = wid * ROWS_T32
    pltpu.sync_copy(dst_hbm.at[pl.ds(base, ROWS_T32)], dst_v)
    pltpu.sync_copy(w_hbm.at[pl.ds(base, ROWS_T32)], w_v)

    def zbody(i, carry):
        deg_v[pl.ds(i * 16, 16)] = jnp.zeros((16,), jnp.float32)
        return carry

    lax.fori_loop(0, NP // 16, zbody, 0)

    def ebody(i, carry):
        r = i // 8
        cc = (i % 8) * 16
        idx = dst_v[r, pl.ds(cc, 16)]
        vals = w_v[r, pl.ds(cc, 16)]
        plsc.addupdate_scatter(deg_v, [idx], vals)
        return carry

    lax.fori_loop(0, ROWS_T32 * 8, ebody, 0)
    pltpu.sync_copy(deg_v, out_hbm.at[wid])


def _prep_body(degp_ref, out_ref):
    deg = jnp.sum(degp_ref[...], axis=0, keepdims=True) + 1.0
    out_ref[...] = lax.rsqrt(deg)


def _round_body(h_hbm, src_hbm, dst_hbm, w_hbm, dinv_hbm, p_out,
                dinv_v, src_b, dst_b, nrm_b, rows0, rows1, acc,
                sem0, sem1, ssem0, ssem1):
    c = lax.axis_index("c")
    s = lax.axis_index("s")
    wid = c * 16 + s
    bufs = (rows0, rows1)
    sems = (sem0, sem1)
    ssems = (ssem0, ssem1)

    pltpu.sync_copy(dinv_hbm, dinv_v)

    def scale_rows(rows_ref, nrows, get_scale):
        @plsc.parallel_loop(0, nrows, 1, unroll=4)
        def rbody(r):
            sc = get_scale(r)
            for k in range(8):
                rows_ref[r, pl.ds(k * 16, 16)] = (
                    rows_ref[r, pl.ds(k * 16, 16)] * sc)

    nbase = s * NROW_T

    @plsc.parallel_loop(0, IB, 1, unroll=4)
    def zbody(r):
        for k in range(8):
            rows0[r, pl.ds(k * 16, 16)] = jnp.zeros((16,), jnp.float32)

    def ibody(b, carry):
        rb = nbase + b * IB
        pltpu.sync_copy(rows0.at[pl.ds(0, IB)], acc.at[pl.ds(rb, IB)])
        return carry

    lax.fori_loop(0, NROW_T // IB, ibody, 0)

    plsc.subcore_barrier()

    def bbody(b, carry):
        ebase = wid * ROWS_T32 + b * EB
        pltpu.sync_copy(src_hbm.at[pl.ds(ebase, EB)], src_b)
        pltpu.sync_copy(dst_hbm.at[pl.ds(ebase, EB)], dst_b)
        pltpu.sync_copy(w_hbm.at[pl.ds(ebase, EB)], nrm_b)

        @plsc.parallel_loop(0, EB, 1, unroll=2)
        def nbody(r):
            for kk in range(8):
                cc = kk * 16
                si = src_b[r, pl.ds(cc, 16)]
                di = dst_b[r, pl.ds(cc, 16)]
                nv = nrm_b[r, pl.ds(cc, 16)]
                nv = (plsc.load_gather(dinv_v, [si]) * nv
                      * plsc.load_gather(dinv_v, [di]))
                nrm_b[r, pl.ds(cc, 16)] = nv

        pltpu.async_copy(h_hbm.at[src_b.at[0]], bufs[0], sems[0])
        pending = [None, None]
        for j in range(EB):
            p = j % 2
            pltpu.make_async_copy(
                h_hbm.at[src_b.at[j]], bufs[p], sems[p]).wait()
            if j + 1 < EB:
                if pending[1 - p] is not None:
                    pending[1 - p].wait()
                    pending[1 - p] = None
                pltpu.async_copy(
                    h_hbm.at[src_b.at[j + 1]], bufs[1 - p], sems[1 - p])

            def nscale(r, j=j):
                return plsc.load_gather(
                    nrm_b,
                    [jnp.full((16,), j, jnp.int32),
                     jnp.full((16,), r, jnp.int32)])

            scale_rows(bufs[p], 128, nscale)
            pending[p] = pltpu.async_copy(
                bufs[p], acc.at[dst_b.at[j]], ssems[p], add=True)
        for p in range(2):
            if pending[p] is not None:
                pending[p].wait()
        return carry

    lax.fori_loop(0, NBLK, bbody, 0)

    plsc.subcore_barrier()

    pltpu.sync_copy(acc.at[pl.ds(nbase, NROW_T)],
                    p_out.at[c].at[pl.ds(nbase, NROW_T)])


def _combine_body(pa_ref, pb_ref, h_ref, dinv_ref, out_ref):
    d2 = (dinv_ref[...] * dinv_ref[...])[:, None]
    out_ref[...] = pa_ref[...] + pb_ref[...] + d2 * h_ref[...]


def _head_body(pa_ref, pb_ref, h_ref, dinv_ref, w_ref, out_ref):
    d2 = (dinv_ref[...] * dinv_ref[...])[:, None]
    h = pa_ref[...] + pb_ref[...] + d2 * h_ref[...]
    g = jnp.dot(h, w_ref[...], preferred_element_type=jnp.float32)
    m = jnp.max(g, axis=1, keepdims=True)
    lse = jnp.log(jnp.sum(jnp.exp(g - m), axis=1, keepdims=True)) + m
    out_ref[...] = g - lse


def kernel(x, edge_index, edge_weight, W):
    src = edge_index[0].astype(jnp.int32)
    dst = edge_index[1].astype(jnp.int32)
    w = edge_weight.astype(jnp.float32)
    e = src.shape[0]

    src2 = jnp.pad(src, (0, EP - e)).reshape(EROWS, 128)
    dst2 = jnp.pad(dst, (0, EP - e)).reshape(EROWS, 128)
    w2 = jnp.pad(w, (0, EP - e)).reshape(EROWS, 128)
    x_pad = jnp.pad(x, ((0, NP - N_NODES), (0, 0)))

    mesh = plsc.VectorSubcoreMesh(core_axis_name="c", subcore_axis_name="s",
                                  num_cores=2, num_subcores=16)
    sc_params = pltpu.CompilerParams(needs_layout_passes=False)

    deg_call = functools.partial(
        pl.kernel, _deg_body, mesh=mesh,
        compiler_params=sc_params,
        out_type=jax.ShapeDtypeStruct((32, NP), jnp.float32),
        scratch_types=[
            pltpu.VMEM((ROWS_T32, 128), jnp.int32),
            pltpu.VMEM((ROWS_T32, 128), jnp.float32),
            pltpu.VMEM((NP,), jnp.float32),
        ])()
    degp = deg_call(dst2, w2)

    dinv = pl.pallas_call(
        _prep_body,
        out_shape=jax.ShapeDtypeStruct((1, NP), jnp.float32),
    )(degp).reshape(NP)

    round_call = functools.partial(
        pl.kernel, _round_body, mesh=mesh,
        compiler_params=sc_params,
        out_type=jax.ShapeDtypeStruct((2, NP, D), jnp.float32),
        scratch_types=[
            pltpu.VMEM((NP,), jnp.float32),
            pltpu.VMEM((EB, 128), jnp.int32),
            pltpu.VMEM((EB, 128), jnp.int32),
            pltpu.VMEM((EB, 128), jnp.float32),
            pltpu.VMEM((128, D), jnp.float32),
            pltpu.VMEM((128, D), jnp.float32),
            pltpu.VMEM_SHARED((NP, D), jnp.float32),
            pltpu.SemaphoreType.DMA,
            pltpu.SemaphoreType.DMA,
            pltpu.SemaphoreType.DMA,
            pltpu.SemaphoreType.DMA,
        ])()

    combine = functools.partial(
        pl.pallas_call, _combine_body,
        grid=(NP // 1024,),
        in_specs=[
            pl.BlockSpec((1024, D), lambda i: (i, 0)),
            pl.BlockSpec((1024, D), lambda i: (i, 0)),
            pl.BlockSpec((1024, D), lambda i: (i, 0)),
            pl.BlockSpec((1024,), lambda i: (i,)),
        ],
        out_specs=pl.BlockSpec((1024, D), lambda i: (i, 0)),
        out_shape=jax.ShapeDtypeStruct((NP, D), jnp.float32),
    )()

    h = x_pad
    parts = None
    for _ in range(K):
        if parts is not None:
            h = combine(parts[0], parts[1], h, dinv)
        p = round_call(h, src2, dst2, w2, dinv)
        parts = (p[0], p[1])

    out = pl.pallas_call(
        _head_body,
        grid=(NP // 1024,),
        in_specs=[
            pl.BlockSpec((1024, D), lambda i: (i, 0)),
            pl.BlockSpec((1024, D), lambda i: (i, 0)),
            pl.BlockSpec((1024, D), lambda i: (i, 0)),
            pl.BlockSpec((1024,), lambda i: (i,)),
            pl.BlockSpec((D, D), lambda i: (0, 0)),
        ],
        out_specs=pl.BlockSpec((1024, D), lambda i: (i, 0)),
        out_shape=jax.ShapeDtypeStruct((NP, D), jnp.float32),
    )(parts[0], parts[1], h, dinv, W)
    return out[:N_NODES]


# speedup vs baseline: 1.1418x; 1.1418x over previous
import functools

import jax
import jax.numpy as jnp
from jax import lax
from jax.experimental import pallas as pl
from jax.experimental.pallas import tpu as pltpu
from jax.experimental.pallas import tpu_sc as plsc

N_NODES = 10000
D = 128
K = 3

NP = 10240
EP = 327680
EROWS = EP // 128
ROWS_T32 = EROWS // 32
NROW_T = NP // 16
IB = 64
EB = 8
RF = 120
RS = 160 - RF


def _deg_body(dst_hbm, w_hbm, out_hbm, dst_v, w_v, deg_v):
    c = lax.axis_index("c")
    s = lax.axis_index("s")
    wid = c * 16 + s
    base = wid * ROWS_T32
    pltpu.sync_copy(dst_hbm.at[pl.ds(base, ROWS_T32)], dst_v)
    pltpu.sync_copy(w_hbm.at[pl.ds(base, ROWS_T32)], w_v)

    def zbody(i, carry):
        deg_v[pl.ds(i * 16, 16)] = jnp.zeros((16,), jnp.float32)
        return carry

    lax.fori_loop(0, NP // 16, zbody, 0)

    def ebody(i, carry):
        r = i // 8
        cc = (i % 8) * 16
        idx = dst_v[r, pl.ds(cc, 16)]
        vals = w_v[r, pl.ds(cc, 16)]
        plsc.addupdate_scatter(deg_v, [idx], vals)
        return carry

    lax.fori_loop(0, ROWS_T32 * 8, ebody, 0)
    pltpu.sync_copy(deg_v, out_hbm.at[wid])


def _prep_body(degp_ref, out_ref):
    deg = jnp.sum(degp_ref[...], axis=0, keepdims=True) + 1.0
    out_ref[...] = lax.rsqrt(deg)


def _round_body(h_hbm, src_hbm, dst_hbm, w_hbm, dinv_hbm, p_out,
                dinv_v, src_b, dst_b, nrm_b, rows0, rows1, acc,
                sem0, sem1, ssem0, ssem1):
    c = lax.axis_index("c")
    s = lax.axis_index("s")
    wid = c * 16 + s
    bufs = (rows0, rows1)
    sems = (sem0, sem1)
    ssems = (ssem0, ssem1)

    pltpu.sync_copy(dinv_hbm, dinv_v)

    def scale_rows(rows_ref, nrows, get_scale):
        @plsc.parallel_loop(0, nrows, 1, unroll=4)
        def rbody(r):
            sc = get_scale(r)
            for k in range(8):
                rows_ref[r, pl.ds(k * 16, 16)] = (
                    rows_ref[r, pl.ds(k * 16, 16)] * sc)

    nbase = s * NROW_T

    @plsc.parallel_loop(0, IB, 1, unroll=4)
    def zbody(r):
        for k in range(8):
            rows0[r, pl.ds(k * 16, 16)] = jnp.zeros((16,), jnp.float32)

    def ibody(b, carry):
        rb = nbase + b * IB
        pltpu.sync_copy(rows0.at[pl.ds(0, IB)], acc.at[pl.ds(rb, IB)])
        return carry

    lax.fori_loop(0, NROW_T // IB, ibody, 0)

    plsc.subcore_barrier()

    tile_base = jnp.where(c == 0, s * RF, 16 * RF + s * RS)
    nblk = jnp.where(c == 0, RF // EB, RS // EB)

    def bbody(b, carry):
        ebase = tile_base + b * EB
        pltpu.sync_copy(src_hbm.at[pl.ds(ebase, EB)], src_b)
        pltpu.sync_copy(dst_hbm.at[pl.ds(ebase, EB)], dst_b)
        pltpu.sync_copy(w_hbm.at[pl.ds(ebase, EB)], nrm_b)

        @plsc.parallel_loop(0, EB, 1, unroll=2)
        def nbody(r):
            for kk in range(8):
                cc = kk * 16
                si = src_b[r, pl.ds(cc, 16)]
                di = dst_b[r, pl.ds(cc, 16)]
                nv = nrm_b[r, pl.ds(cc, 16)]
                nv = (plsc.load_gather(dinv_v, [si]) * nv
                      * plsc.load_gather(dinv_v, [di]))
                nrm_b[r, pl.ds(cc, 16)] = nv

        pltpu.async_copy(h_hbm.at[src_b.at[0]], bufs[0], sems[0])
        pending = [None, None]
        for j in range(EB):
            p = j % 2
            pltpu.make_async_copy(
                h_hbm.at[src_b.at[j]], bufs[p], sems[p]).wait()
            if j + 1 < EB:
                if pending[1 - p] is not None:
                    pending[1 - p].wait()
                    pending[1 - p] = None
                pltpu.async_copy(
                    h_hbm.at[src_b.at[j + 1]], bufs[1 - p], sems[1 - p])

            def nscale(r, j=j):
                return plsc.load_gather(
                    nrm_b,
                    [jnp.full((16,), j, jnp.int32),
                     jnp.full((16,), r, jnp.int32)])

            scale_rows(bufs[p], 128, nscale)
            pending[p] = pltpu.async_copy(
                bufs[p], acc.at[dst_b.at[j]], ssems[p], add=True)
        for p in range(2):
            if pending[p] is not None:
                pending[p].wait()
        return carry

    lax.fori_loop(0, nblk, bbody, 0)

    plsc.subcore_barrier()

    pltpu.sync_copy(acc.at[pl.ds(nbase, NROW_T)],
                    p_out.at[c].at[pl.ds(nbase, NROW_T)])


def _combine_body(pa_ref, pb_ref, h_ref, dinv_ref, out_ref):
    d2 = (dinv_ref[...] * dinv_ref[...])[:, None]
    out_ref[...] = pa_ref[...] + pb_ref[...] + d2 * h_ref[...]


def _head_body(pa_ref, pb_ref, h_ref, dinv_ref, w_ref, out_ref):
    d2 = (dinv_ref[...] * dinv_ref[...])[:, None]
    h = pa_ref[...] + pb_ref[...] + d2 * h_ref[...]
    g = jnp.dot(h, w_ref[...], preferred_element_type=jnp.float32)
    m = jnp.max(g, axis=1, keepdims=True)
    lse = jnp.log(jnp.sum(jnp.exp(g - m), axis=1, keepdims=True)) + m
    out_ref[...] = g - lse


def kernel(x, edge_index, edge_weight, W):
    src = edge_index[0].astype(jnp.int32)
    dst = edge_index[1].astype(jnp.int32)
    w = edge_weight.astype(jnp.float32)
    e = src.shape[0]

    src2 = jnp.pad(src, (0, EP - e)).reshape(EROWS, 128)
    dst2 = jnp.pad(dst, (0, EP - e)).reshape(EROWS, 128)
    w2 = jnp.pad(w, (0, EP - e)).reshape(EROWS, 128)
    x_pad = jnp.pad(x, ((0, NP - N_NODES), (0, 0)))

    mesh = plsc.VectorSubcoreMesh(core_axis_name="c", subcore_axis_name="s",
                                  num_cores=2, num_subcores=16)
    sc_params = pltpu.CompilerParams(needs_layout_passes=False)

    deg_call = functools.partial(
        pl.kernel, _deg_body, mesh=mesh,
        compiler_params=sc_params,
        out_type=jax.ShapeDtypeStruct((32, NP), jnp.float32),
        scratch_types=[
            pltpu.VMEM((ROWS_T32, 128), jnp.int32),
            pltpu.VMEM((ROWS_T32, 128), jnp.float32),
            pltpu.VMEM((NP,), jnp.float32),
        ])()
    degp = deg_call(dst2, w2)

    dinv = pl.pallas_call(
        _prep_body,
        out_shape=jax.ShapeDtypeStruct((1, NP), jnp.float32),
    )(degp).reshape(NP)

    round_call = functools.partial(
        pl.kernel, _round_body, mesh=mesh,
        compiler_params=sc_params,
        out_type=jax.ShapeDtypeStruct((2, NP, D), jnp.float32),
        scratch_types=[
            pltpu.VMEM((NP,), jnp.float32),
            pltpu.VMEM((EB, 128), jnp.int32),
            pltpu.VMEM((EB, 128), jnp.int32),
            pltpu.VMEM((EB, 128), jnp.float32),
            pltpu.VMEM((128, D), jnp.float32),
            pltpu.VMEM((128, D), jnp.float32),
            pltpu.VMEM_SHARED((NP, D), jnp.float32),
            pltpu.SemaphoreType.DMA,
            pltpu.SemaphoreType.DMA,
            pltpu.SemaphoreType.DMA,
            pltpu.SemaphoreType.DMA,
        ])()

    combine = functools.partial(
        pl.pallas_call, _combine_body,
        grid=(NP // 1024,),
        in_specs=[
            pl.BlockSpec((1024, D), lambda i: (i, 0)),
            pl.BlockSpec((1024, D), lambda i: (i, 0)),
            pl.BlockSpec((1024, D), lambda i: (i, 0)),
            pl.BlockSpec((1024,), lambda i: (i,)),
        ],
        out_specs=pl.BlockSpec((1024, D), lambda i: (i, 0)),
        out_shape=jax.ShapeDtypeStruct((NP, D), jnp.float32),
    )()

    h = x_pad
    parts = None
    for _ in range(K):
        if parts is not None:
            h = combine(parts[0], parts[1], h, dinv)
        p = round_call(h, src2, dst2, w2, dinv)
        parts = (p[0], p[1])

    out = pl.pallas_call(
        _head_body,
        grid=(NP // 1024,),
        in_specs=[
            pl.BlockSpec((1024, D), lambda i: (i, 0)),
            pl.BlockSpec((1024, D), lambda i: (i, 0)),
            pl.BlockSpec((1024, D), lambda i: (i, 0)),
            pl.BlockSpec((1024,), lambda i: (i,)),
            pl.BlockSpec((D, D), lambda i: (0, 0)),
        ],
        out_specs=pl.BlockSpec((1024, D), lambda i: (i, 0)),
        out_shape=jax.ShapeDtypeStruct((NP, D), jnp.float32),
    )(parts[0], parts[1], h, dinv, W)
    return out[:N_NODES]
